# baseline (speedup 1.0000x reference)
import jax
import jax.numpy as jnp
from jax import lax
from jax.experimental import pallas as pl
from jax.experimental.pallas import tpu as pltpu
from jax.experimental.pallas import tpu_sc as plsc

B = 16384
D = 64
NC = 2
NS = 16
L = 16
NW = NC * NS
BPW = B // NW
CH = 32
NG = BPW // CH


def _fac_body(uid_hbm, mid_hbm, users_hbm, movies_hbm, ub_hbm, mb_hbm,
              out_hbm,
              uidx_v, midx_v, ubuf, mbuf, ubb, mbb, out_v, sem):
    c = lax.axis_index("c")
    s = lax.axis_index("s")
    wid = s * NC + c

    pltpu.sync_copy(uid_hbm.at[wid], uidx_v)
    pltpu.sync_copy(mid_hbm.at[wid], midx_v)

    lane = lax.iota(jnp.int32, L)

    def chunk(g, carry):
        vecs = []
        for h in range(CH // L):
            vecs.append((uidx_v[pl.ds(g * CH + h * L, L)],
                         midx_v[pl.ds(g * CH + h * L, L)]))

        for h, (uvec, mvec) in enumerate(vecs):
            for i in range(L):
                p = h * L + i
                ui = uvec[i]
                mi = mvec[i]
                pltpu.async_copy(users_hbm.at[ui], ubuf.at[p], sem)
                pltpu.async_copy(movies_hbm.at[mi], mbuf.at[p], sem)
                pltpu.async_copy(ub_hbm.at[ui], ubb.at[p], sem)
                pltpu.async_copy(mb_hbm.at[mi], mbb.at[p], sem)
        pltpu.make_async_copy(users_hbm.at[pl.ds(0, CH)], ubuf, sem).wait()
        pltpu.make_async_copy(movies_hbm.at[pl.ds(0, CH)], mbuf, sem).wait()
        pltpu.make_async_copy(ub_hbm.at[pl.ds(0, CH)], ubb, sem).wait()
        pltpu.make_async_copy(mb_hbm.at[pl.ds(0, CH)], mbb, sem).wait()

        zero = jnp.zeros((L,), jnp.int32)
        for h in range(CH // L):
            pv = h * L + lane
            acc = (plsc.load_gather(ubb, [pv, zero])
                   + plsc.load_gather(mbb, [pv, zero]))
            for k in range(D):
                kv = jnp.full((L,), k, jnp.int32)
                u = plsc.load_gather(ubuf, [pv, kv])
                m = plsc.load_gather(mbuf, [pv, kv])
                acc = acc + u * m
            out_v[pl.ds(g * CH + h * L, L)] = acc
        return carry

    lax.fori_loop(0, NG, chunk, 0)

    pltpu.sync_copy(out_v, out_hbm.at[pl.ds(wid * BPW, BPW)])


def kernel(user_ids, movie_ids, users, movies, user_bias, movie_bias):
    uid = user_ids.astype(jnp.int32).reshape(NW, BPW)
    mid = movie_ids.astype(jnp.int32).reshape(NW, BPW)

    mesh = plsc.VectorSubcoreMesh(core_axis_name="c", subcore_axis_name="s")
    fn = pl.kernel(
        _fac_body,
        out_type=jax.ShapeDtypeStruct((B,), jnp.float32),
        mesh=mesh,
        compiler_params=pltpu.CompilerParams(
            needs_layout_passes=False, use_tc_tiling_on_sc=True),
        scratch_types=[
            pltpu.VMEM((BPW,), jnp.int32),
            pltpu.VMEM((BPW,), jnp.int32),
            pltpu.VMEM((CH, D), jnp.float32),
            pltpu.VMEM((CH, D), jnp.float32),
            pltpu.VMEM((CH, 1), jnp.float32),
            pltpu.VMEM((CH, 1), jnp.float32),
            pltpu.VMEM((BPW,), jnp.float32),
            pltpu.SemaphoreType.DMA,
        ],
    )
    return fn(uid, mid, users, movies, user_bias, movie_bias)


# speedup vs baseline: 1.0021x; 1.0021x over previous
import jax
import jax.numpy as jnp
from jax import lax
from jax.experimental import pallas as pl
from jax.experimental.pallas import tpu as pltpu
from jax.experimental.pallas import tpu_sc as plsc

B = 16384
D = 64
NC = 2
NS = 16
L = 16
NW = NC * NS
BPW = B // NW
CH = 32
NG = BPW // CH


NSEM = 8


def _fac_body(uid_hbm, mid_hbm, users_hbm, movies_hbm, ub_hbm, mb_hbm,
              out_hbm,
              uidx_v, midx_v, ubuf, mbuf, ubb, mbb, out_v, *sems):
    sem = sems[0]
    c = lax.axis_index("c")
    s = lax.axis_index("s")
    wid = s * NC + c

    pltpu.sync_copy(uid_hbm.at[wid], uidx_v)
    pltpu.sync_copy(mid_hbm.at[wid], midx_v)

    lane = lax.iota(jnp.int32, L)

    def chunk(g, carry):
        vecs = []
        for h in range(CH // L):
            vecs.append((uidx_v[pl.ds(g * CH + h * L, L)],
                         midx_v[pl.ds(g * CH + h * L, L)]))

        for h, (uvec, mvec) in enumerate(vecs):
            for i in range(L):
                p = h * L + i
                ui = uvec[i]
                mi = mvec[i]
                sq = sems[p % NSEM]
                pltpu.async_copy(users_hbm.at[ui], ubuf.at[p], sq)
                pltpu.async_copy(movies_hbm.at[mi], mbuf.at[p], sq)
                pltpu.async_copy(ub_hbm.at[ui], ubb.at[p], sq)
                pltpu.async_copy(mb_hbm.at[mi], mbb.at[p], sq)
        npq = CH // NSEM
        for s in range(NSEM):
            sq = sems[s]
            pltpu.make_async_copy(
                users_hbm.at[pl.ds(0, npq)], ubuf.at[pl.ds(0, npq)], sq).wait()
            pltpu.make_async_copy(
                movies_hbm.at[pl.ds(0, npq)], mbuf.at[pl.ds(0, npq)], sq).wait()
            pltpu.make_async_copy(
                ub_hbm.at[pl.ds(0, npq)], ubb.at[pl.ds(0, npq)], sq).wait()
            pltpu.make_async_copy(
                mb_hbm.at[pl.ds(0, npq)], mbb.at[pl.ds(0, npq)], sq).wait()

        zero = jnp.zeros((L,), jnp.int32)
        for h in range(CH // L):
            pv = h * L + lane
            acc = (plsc.load_gather(ubb, [pv, zero])
                   + plsc.load_gather(mbb, [pv, zero]))
            for k in range(D):
                kv = jnp.full((L,), k, jnp.int32)
                u = plsc.load_gather(ubuf, [pv, kv])
                m = plsc.load_gather(mbuf, [pv, kv])
                acc = acc + u * m
            out_v[pl.ds(g * CH + h * L, L)] = acc
        return carry

    lax.fori_loop(0, NG, chunk, 0)

    pltpu.sync_copy(out_v, out_hbm.at[pl.ds(wid * BPW, BPW)])


def kernel(user_ids, movie_ids, users, movies, user_bias, movie_bias):
    uid = user_ids.astype(jnp.int32).reshape(NW, BPW)
    mid = movie_ids.astype(jnp.int32).reshape(NW, BPW)

    mesh = plsc.VectorSubcoreMesh(core_axis_name="c", subcore_axis_name="s")
    fn = pl.kernel(
        _fac_body,
        out_type=jax.ShapeDtypeStruct((B,), jnp.float32),
        mesh=mesh,
        compiler_params=pltpu.CompilerParams(
            needs_layout_passes=False, use_tc_tiling_on_sc=True),
        scratch_types=[
            pltpu.VMEM((BPW,), jnp.int32),
            pltpu.VMEM((BPW,), jnp.int32),
            pltpu.VMEM((CH, D), jnp.float32),
            pltpu.VMEM((CH, D), jnp.float32),
            pltpu.VMEM((CH, 1), jnp.float32),
            pltpu.VMEM((CH, 1), jnp.float32),
            pltpu.VMEM((BPW,), jnp.float32),
        ] + [pltpu.SemaphoreType.DMA] * NSEM,
    )
    return fn(uid, mid, users, movies, user_bias, movie_bias)
